# baseline (speedup 1.0000x reference)
import functools

import jax
import jax.numpy as jnp
from jax import lax
from jax.experimental import pallas as pl
from jax.experimental.pallas import tpu as pltpu
from jax.experimental.pallas import tpu_sc as plsc

_B = 16384
_L = 200
_LW = _L // 4
_VPAD = 128
_NW = 32
_RPW = _B // _NW
_NG = _RPW // 16
_PAR_EMB = 32
_PAR_N = _PAR_EMB + 4 * _VPAD


def _sc_body(x_hbm, par_hbm, out_hbm,
             xb0_v, xb1_v, par_v, m0_v, m1_v, out_v, sem0, sem1):
    wid = lax.axis_index("s") * 2 + lax.axis_index("c")
    base = wid * _RPW
    bufs = ((xb0_v, sem0), (xb1_v, sem1))

    def chunk_copy(g, slot):
        buf, sem = bufs[slot]
        return pltpu.make_async_copy(
            x_hbm.at[pl.ds(base + g * 16, 16)], buf, sem)

    chunk_copy(0, 0).start()
    chunk_copy(1, 1).start()
    pltpu.sync_copy(par_hbm, par_v)

    iota = lax.iota(jnp.int32, 16)

    def bcast_par(i):
        return plsc.load_gather(par_v, [jnp.full((16,), i, jnp.int32)])

    wv = [[bcast_par(1 + 4 * o + d) for d in range(4)] for o in range(2)]
    bv = [bcast_par(9 + o) for o in range(2)]

    for c in range(_VPAD // 16):
        vidx = iota + (16 * c)
        e = [plsc.load_gather(par_v, [vidx + (_PAR_EMB + d * _VPAD)])
             for d in range(4)]
        m0 = e[0] * wv[0][0] + e[1] * wv[0][1] + e[2] * wv[0][2] + e[3] * wv[0][3]
        m1 = e[0] * wv[1][0] + e[1] * wv[1][1] + e[2] * wv[1][2] + e[3] * wv[1][3]
        m0_v[pl.ds(16 * c, 16)] = m0
        m1_v[pl.ds(16 * c, 16)] = m1

    scale = jnp.full((16,), 1.0 / _L, jnp.float32)
    mask_ff = jnp.full((16,), 0xFF, jnp.int32)

    def do_group(g, slot):
        buf, _ = bufs[slot]
        chunk_copy(g, slot).wait()

        def step(i, carry):
            a0, a1, wcol = carry
            for _ in range(2):
                pw = plsc.load_gather(buf, [iota, wcol])
                for xi in (pw & mask_ff,
                           lax.shift_right_logical(pw, 8) & mask_ff,
                           lax.shift_right_logical(pw, 16) & mask_ff,
                           lax.shift_right_logical(pw, 24)):
                    a0 = a0 + plsc.load_gather(m0_v, [xi])
                    a1 = a1 + plsc.load_gather(m1_v, [xi])
                wcol = wcol + 1
            return a0, a1, wcol

        zero = jnp.zeros((16,), jnp.float32)
        a0, a1, _ = lax.fori_loop(
            0, _LW // 2, step, (zero, zero, jnp.zeros((16,), jnp.int32)))
        row16 = iota + g * 16
        plsc.store_scatter(out_v, [row16, jnp.zeros((16,), jnp.int32)],
                           a0 * scale + bv[0])
        plsc.store_scatter(out_v, [row16, jnp.full((16,), 1, jnp.int32)],
                           a1 * scale + bv[1])

        @pl.when(g + 2 < _NG)
        def _prefetch():
            chunk_copy(g + 2, slot).start()

    def pair(p, _):
        do_group(2 * p, 0)
        do_group(2 * p + 1, 1)
        return 0

    lax.fori_loop(0, _NG // 2, pair, 0)
    pltpu.sync_copy(out_v, out_hbm.at[pl.ds(base, _RPW)])


_toy_sc = functools.partial(
    pl.kernel,
    mesh=plsc.VectorSubcoreMesh(core_axis_name="c", subcore_axis_name="s"),
    out_type=jax.ShapeDtypeStruct((_B, 2), jnp.float32),
    compiler_params=pltpu.CompilerParams(needs_layout_passes=False),
    scratch_types=[
        pltpu.VMEM((16, _LW), jnp.int32),
        pltpu.VMEM((16, _LW), jnp.int32),
        pltpu.VMEM((_PAR_N,), jnp.float32),
        pltpu.VMEM((_VPAD,), jnp.float32),
        pltpu.VMEM((_VPAD,), jnp.float32),
        pltpu.VMEM((_RPW, 2), jnp.float32),
        pltpu.SemaphoreType.DMA,
        pltpu.SemaphoreType.DMA,
    ],
)(_sc_body)


def kernel(x, emb, W, b):
    x32 = x.astype(jnp.int32)
    xp = (x32[:, 0::4] | (x32[:, 1::4] << 8) | (x32[:, 2::4] << 16)
          | (x32[:, 3::4] << 24))
    embt = jnp.zeros((4, _VPAD), jnp.float32).at[:, :100].set(
        emb.astype(jnp.float32).T)
    par = jnp.concatenate(
        [jnp.zeros((1,), jnp.float32), W.reshape(-1).astype(jnp.float32),
         b.astype(jnp.float32), jnp.zeros((_PAR_EMB - 11,), jnp.float32),
         embt.reshape(-1)])
    return _toy_sc(xp, par)


# speedup vs baseline: 2.4720x; 2.4720x over previous
import functools

import jax
import jax.numpy as jnp
from jax import lax
from jax.experimental import pallas as pl
from jax.experimental.pallas import tpu as pltpu
from jax.experimental.pallas import tpu_sc as plsc

_B = 16384
_L = 200
_LP = 208
_VPAD = 128
_NW = 32
_RPW = _B // _NW
_NG = _RPW // 16
_PAR_EMB = 32
_PAR_N = _PAR_EMB + 4 * _VPAD


def _sc_body(x_hbm, par_hbm, out_hbm,
             xb0_v, xb1_v, par_v, m0_v, m1_v, t0_v, t1_v, out_v, sem0, sem1):
    wid = lax.axis_index("s") * 2 + lax.axis_index("c")
    base = wid * _RPW
    bufs = ((xb0_v, sem0), (xb1_v, sem1))

    def chunk_copy(g, slot):
        buf, sem = bufs[slot]
        return pltpu.make_async_copy(
            x_hbm.at[pl.ds(base + g * 16, 16)], buf, sem)

    chunk_copy(0, 0).start()
    chunk_copy(1, 1).start()
    pltpu.sync_copy(par_hbm, par_v)

    iota = lax.iota(jnp.int32, 16)

    def bcast_par(i):
        return plsc.load_gather(par_v, [jnp.full((16,), i, jnp.int32)])

    wv = [[bcast_par(1 + 4 * o + d) for d in range(4)] for o in range(2)]

    for c in range(_VPAD // 16):
        vidx = iota + (16 * c)
        e = [plsc.load_gather(par_v, [vidx + (_PAR_EMB + d * _VPAD)])
             for d in range(4)]
        m0 = e[0] * wv[0][0] + e[1] * wv[0][1] + e[2] * wv[0][2] + e[3] * wv[0][3]
        m1 = e[0] * wv[1][0] + e[1] * wv[1][1] + e[2] * wv[1][2] + e[3] * wv[1][3]
        m0_v[pl.ds(16 * c, 16)] = m0
        m1_v[pl.ds(16 * c, 16)] = m1

    bv = [bcast_par(9 + o) for o in range(2)]
    scale = jnp.full((16,), 1.0 / _L, jnp.float32)
    iota16 = iota * 16

    def do_group(g, slot):
        buf, _ = bufs[slot]
        chunk_copy(g, slot).wait()

        for r in range(16):
            a0 = jnp.zeros((16,), jnp.float32)
            a1 = jnp.zeros((16,), jnp.float32)
            for k in range(13):
                if k < 12:
                    xi = buf[r, pl.ds(16 * k, 16)]
                else:
                    xi = jnp.where(iota < 8, 100, buf[r, pl.ds(_L - 16, 16)])
                a0 = a0 + plsc.load_gather(m0_v, [xi])
                a1 = a1 + plsc.load_gather(m1_v, [xi])
            t0_v[pl.ds(16 * r, 16)] = a0
            t1_v[pl.ds(16 * r, 16)] = a1

        s0 = jnp.zeros((16,), jnp.float32)
        s1 = jnp.zeros((16,), jnp.float32)
        for j in range(16):
            s0 = s0 + plsc.load_gather(t0_v, [iota16 + j])
            s1 = s1 + plsc.load_gather(t1_v, [iota16 + j])

        row16 = iota + g * 16
        plsc.store_scatter(out_v, [row16, jnp.zeros((16,), jnp.int32)],
                           s0 * scale + bv[0])
        plsc.store_scatter(out_v, [row16, jnp.full((16,), 1, jnp.int32)],
                           s1 * scale + bv[1])

        @pl.when(g + 2 < _NG)
        def _prefetch():
            chunk_copy(g + 2, slot).start()

    def pair(p, _):
        do_group(2 * p, 0)
        do_group(2 * p + 1, 1)
        return 0

    lax.fori_loop(0, _NG // 2, pair, 0)
    pltpu.sync_copy(out_v, out_hbm.at[pl.ds(base, _RPW)])


_toy_sc = functools.partial(
    pl.kernel,
    mesh=plsc.VectorSubcoreMesh(core_axis_name="c", subcore_axis_name="s"),
    out_type=jax.ShapeDtypeStruct((_B, 2), jnp.float32),
    compiler_params=pltpu.CompilerParams(needs_layout_passes=False),
    scratch_types=[
        pltpu.VMEM((16, _L), jnp.int32),
        pltpu.VMEM((16, _L), jnp.int32),
        pltpu.VMEM((_PAR_N,), jnp.float32),
        pltpu.VMEM((_VPAD,), jnp.float32),
        pltpu.VMEM((_VPAD,), jnp.float32),
        pltpu.VMEM((256,), jnp.float32),
        pltpu.VMEM((256,), jnp.float32),
        pltpu.VMEM((_RPW, 2), jnp.float32),
        pltpu.SemaphoreType.DMA,
        pltpu.SemaphoreType.DMA,
    ],
)(_sc_body)


def kernel(x, emb, W, b):
    x32 = x.astype(jnp.int32)
    embt = jnp.zeros((4, _VPAD), jnp.float32).at[:, :100].set(
        emb.astype(jnp.float32).T)
    par = jnp.concatenate(
        [jnp.zeros((1,), jnp.float32), W.reshape(-1).astype(jnp.float32),
         b.astype(jnp.float32), jnp.zeros((_PAR_EMB - 11,), jnp.float32),
         embt.reshape(-1)])
    return _toy_sc(x32, par)


# speedup vs baseline: 2.4873x; 1.0062x over previous
import functools

import jax
import jax.numpy as jnp
from jax import lax
from jax.experimental import pallas as pl
from jax.experimental.pallas import tpu as pltpu
from jax.experimental.pallas import tpu_sc as plsc

_B = 16384
_L = 200
_VPAD = 128
_NW = 32
_RPW = _B // _NW
_NG = _RPW // 16
_PAR_EMB = 32
_PAR_N = _PAR_EMB + 4 * _VPAD


def _sc_body(x_hbm, par_hbm, out_hbm,
             xb0_v, xb1_v, par_v, m0_v, m1_v, t0_v, t1_v, out_v, sem0, sem1):
    wid = lax.axis_index("s") * 2 + lax.axis_index("c")
    base = wid * _RPW
    bufs = ((xb0_v, sem0), (xb1_v, sem1))

    def chunk_copy(g, slot):
        buf, sem = bufs[slot]
        return pltpu.make_async_copy(
            x_hbm.at[pl.ds(base + g * 16, 16)], buf, sem)

    chunk_copy(0, 0).start()
    chunk_copy(1, 1).start()
    pltpu.sync_copy(par_hbm, par_v)

    iota = lax.iota(jnp.int32, 16)

    def bcast_par(i):
        return plsc.load_gather(par_v, [jnp.full((16,), i, jnp.int32)])

    wv = [[bcast_par(1 + 4 * o + d) for d in range(4)] for o in range(2)]

    for c in range(_VPAD // 16):
        vidx = iota + (16 * c)
        e = [plsc.load_gather(par_v, [vidx + (_PAR_EMB + d * _VPAD)])
             for d in range(4)]
        m0 = e[0] * wv[0][0] + e[1] * wv[0][1] + e[2] * wv[0][2] + e[3] * wv[0][3]
        m1 = e[0] * wv[1][0] + e[1] * wv[1][1] + e[2] * wv[1][2] + e[3] * wv[1][3]
        m0_v[pl.ds(16 * c, 16)] = m0
        m1_v[pl.ds(16 * c, 16)] = m1

    bv = [bcast_par(9 + o) for o in range(2)]
    scale = jnp.full((16,), 1.0 / _L, jnp.float32)
    iota16 = iota * 16

    def do_group(g, slot):
        buf, _ = bufs[slot]
        chunk_copy(g, slot).wait()

        for r in range(16):
            a0 = jnp.zeros((16,), jnp.float32)
            a1 = jnp.zeros((16,), jnp.float32)
            for k in range(13):
                if k < 12:
                    xi = buf[r, pl.ds(16 * k, 16)]
                else:
                    xi = jnp.where(iota < 8, 100, buf[r, pl.ds(_L - 16, 16)])
                a0 = a0 + plsc.load_gather(m0_v, [xi])
                a1 = a1 + plsc.load_gather(m1_v, [xi])
            t0_v[pl.ds(16 * r, 16)] = a0
            t1_v[pl.ds(16 * r, 16)] = a1

        s0 = jnp.zeros((16,), jnp.float32)
        s1 = jnp.zeros((16,), jnp.float32)
        for j in range(16):
            s0 = s0 + plsc.load_gather(t0_v, [iota16 + j])
            s1 = s1 + plsc.load_gather(t1_v, [iota16 + j])

        row16 = iota + g * 16
        plsc.store_scatter(out_v, [row16, jnp.zeros((16,), jnp.int32)],
                           s0 * scale + bv[0])
        plsc.store_scatter(out_v, [row16, jnp.full((16,), 1, jnp.int32)],
                           s1 * scale + bv[1])

        @pl.when(g + 2 < _NG)
        def _prefetch():
            chunk_copy(g + 2, slot).start()

    def pair(p, _):
        do_group(2 * p, 0)
        do_group(2 * p + 1, 1)
        return 0

    lax.fori_loop(0, _NG // 2, pair, 0)
    pltpu.sync_copy(out_v, out_hbm.at[pl.ds(base, _RPW)])


_toy_sc = functools.partial(
    pl.kernel,
    mesh=plsc.VectorSubcoreMesh(core_axis_name="c", subcore_axis_name="s"),
    out_type=jax.ShapeDtypeStruct((_B, 2), jnp.float32),
    compiler_params=pltpu.CompilerParams(needs_layout_passes=False),
    scratch_types=[
        pltpu.VMEM((16, _L), jnp.int32),
        pltpu.VMEM((16, _L), jnp.int32),
        pltpu.VMEM((_PAR_N,), jnp.float32),
        pltpu.VMEM((_VPAD,), jnp.float32),
        pltpu.VMEM((_VPAD,), jnp.float32),
        pltpu.VMEM((256,), jnp.float32),
        pltpu.VMEM((256,), jnp.float32),
        pltpu.VMEM((_RPW, 2), jnp.float32),
        pltpu.SemaphoreType.DMA,
        pltpu.SemaphoreType.DMA,
    ],
)(_sc_body)


def kernel(x, emb, W, b):
    x32 = x.astype(jnp.int32)
    embt = jnp.zeros((4, _VPAD), jnp.float32).at[:, :100].set(
        emb.astype(jnp.float32).T)
    par = jnp.concatenate(
        [jnp.zeros((1,), jnp.float32), W.reshape(-1).astype(jnp.float32),
         b.astype(jnp.float32), jnp.zeros((_PAR_EMB - 11,), jnp.float32),
         embt.reshape(-1)])
    return _toy_sc(x32, par)
